# baseline (speedup 1.0000x reference)
import jax
import jax.numpy as jnp
from jax.experimental import pallas as pl
from jax.experimental.pallas import tpu as pltpu

B, P, D = 64, 1024, 192
FR = B * P * D // 1024
TR = P * D // 1024


def _body(x_ref, t_ref, o_ref):
    o_ref[...] = x_ref[...] + t_ref[...]


def kernel(x, pos_table):
    out = pl.pallas_call(
        _body,
        out_shape=jax.ShapeDtypeStruct((FR, 1024), jnp.float32),
        grid=(FR // TR,),
        in_specs=[
            pl.BlockSpec((TR, 1024), lambda i: (i, 0)),
            pl.BlockSpec((TR, 1024), lambda i: (0, 0)),
        ],
        out_specs=pl.BlockSpec((TR, 1024), lambda i: (i, 0)),
        compiler_params=pltpu.CompilerParams(
            dimension_semantics=("arbitrary",),
        ),
    )(x.reshape(FR, 1024), pos_table.reshape(TR, 1024))
    return out.reshape(B, P, D)


# speedup vs baseline: 1.6525x; 1.6525x over previous
import jax
import jax.numpy as jnp
from jax.experimental import pallas as pl
from jax.experimental.pallas import tpu as pltpu

B, P, D = 64, 1024, 192
NBB = 1


def _body(x_ref, t_ref, o_ref):
    o_ref[...] = x_ref[...] + t_ref[...][None, :, :]


def kernel(x, pos_table):
    return pl.pallas_call(
        _body,
        out_shape=jax.ShapeDtypeStruct((B, P, D), jnp.float32),
        grid=(B // NBB,),
        in_specs=[
            pl.BlockSpec((NBB, P, D), lambda i: (i, 0, 0)),
            pl.BlockSpec((P, D), lambda i: (0, 0)),
        ],
        out_specs=pl.BlockSpec((NBB, P, D), lambda i: (i, 0, 0)),
        compiler_params=pltpu.CompilerParams(
            dimension_semantics=("arbitrary",),
        ),
    )(x, pos_table)


# speedup vs baseline: 1.9115x; 1.1568x over previous
import jax
import jax.numpy as jnp
from jax.experimental import pallas as pl
from jax.experimental.pallas import tpu as pltpu

B, P, D = 64, 1024, 192
NBB = 4


def _body(x_ref, t_ref, o_ref):
    o_ref[...] = x_ref[...] + t_ref[...][None, :, :]


def kernel(x, pos_table):
    return pl.pallas_call(
        _body,
        out_shape=jax.ShapeDtypeStruct((B, P, D), jnp.float32),
        grid=(B // NBB,),
        in_specs=[
            pl.BlockSpec((NBB, P, D), lambda i: (i, 0, 0)),
            pl.BlockSpec((P, D), lambda i: (0, 0)),
        ],
        out_specs=pl.BlockSpec((NBB, P, D), lambda i: (i, 0, 0)),
        compiler_params=pltpu.CompilerParams(
            dimension_semantics=("arbitrary",),
        ),
    )(x, pos_table)


# speedup vs baseline: 8.7682x; 4.5870x over previous
import jax
import jax.numpy as jnp
from jax.experimental import pallas as pl
from jax.experimental.pallas import tpu as pltpu

B, P, D = 64, 1024, 192
NBB = 4


def _body(x_ref, t_ref, o_ref):
    o_ref[...] = x_ref[...] + t_ref[...][None, :, :]


def kernel(x, pos_table):
    xt = jnp.transpose(x, (0, 2, 1))
    tt = jnp.transpose(pos_table, (1, 0))
    ot = pl.pallas_call(
        _body,
        out_shape=jax.ShapeDtypeStruct((B, D, P), jnp.float32),
        grid=(B // NBB,),
        in_specs=[
            pl.BlockSpec((NBB, D, P), lambda i: (i, 0, 0)),
            pl.BlockSpec((D, P), lambda i: (0, 0)),
        ],
        out_specs=pl.BlockSpec((NBB, D, P), lambda i: (i, 0, 0)),
        compiler_params=pltpu.CompilerParams(
            dimension_semantics=("arbitrary",),
        ),
    )(xt, tt)
    return jnp.transpose(ot, (0, 2, 1))


# speedup vs baseline: 9.2219x; 1.0517x over previous
import jax
import jax.numpy as jnp
from jax.experimental import pallas as pl
from jax.experimental.pallas import tpu as pltpu

B, P, D = 64, 1024, 192
NBB = 8


def _body(x_ref, t_ref, o_ref):
    o_ref[...] = x_ref[...] + t_ref[...][None, :, :]


def kernel(x, pos_table):
    xt = jnp.transpose(x, (0, 2, 1))
    tt = jnp.transpose(pos_table, (1, 0))
    ot = pl.pallas_call(
        _body,
        out_shape=jax.ShapeDtypeStruct((B, D, P), jnp.float32),
        grid=(B // NBB,),
        in_specs=[
            pl.BlockSpec((NBB, D, P), lambda i: (i, 0, 0)),
            pl.BlockSpec((D, P), lambda i: (0, 0)),
        ],
        out_specs=pl.BlockSpec((NBB, D, P), lambda i: (i, 0, 0)),
        compiler_params=pltpu.CompilerParams(
            dimension_semantics=("arbitrary",),
        ),
    )(xt, tt)
    return jnp.transpose(ot, (0, 2, 1))


# speedup vs baseline: 9.6951x; 1.0513x over previous
import jax
import jax.numpy as jnp
from jax.experimental import pallas as pl
from jax.experimental.pallas import tpu as pltpu

B, P, D = 64, 1024, 192
NBB = 16


def _body(x_ref, t_ref, o_ref):
    o_ref[...] = x_ref[...] + t_ref[...][None, :, :]


def kernel(x, pos_table):
    xt = jnp.transpose(x, (0, 2, 1))
    tt = jnp.transpose(pos_table, (1, 0))
    ot = pl.pallas_call(
        _body,
        out_shape=jax.ShapeDtypeStruct((B, D, P), jnp.float32),
        grid=(B // NBB,),
        in_specs=[
            pl.BlockSpec((NBB, D, P), lambda i: (i, 0, 0)),
            pl.BlockSpec((D, P), lambda i: (0, 0)),
        ],
        out_specs=pl.BlockSpec((NBB, D, P), lambda i: (i, 0, 0)),
        compiler_params=pltpu.CompilerParams(
            dimension_semantics=("arbitrary",),
        ),
    )(xt, tt)
    return jnp.transpose(ot, (0, 2, 1))
